# baseline (speedup 1.0000x reference)
import functools

import jax
import jax.numpy as jnp
from jax import lax
from jax.experimental import pallas as pl
from jax.experimental.pallas import tpu as pltpu
from jax.experimental.pallas import tpu_sc as plsc

_B = 256
_IB = 512
_LANES = 128


_NUM_E = 8


def _ffn_body(be_r, used_r, x_r, s_r, gw_r, uw_r, dw_r, o_r):
    g = pl.program_id(0)
    j = pl.program_id(1)
    is_active = g < used_r[0]

    @pl.when(jnp.logical_and(is_active, j == 0))
    def _():
        o_r[...] = jnp.zeros_like(o_r)

    @pl.when(is_active)
    def _():
        x = x_r[...].astype(jnp.bfloat16)
        s = s_r[...]
        gate = lax.dot_general(x, gw_r[0].astype(jnp.bfloat16),
                               (((1,), (0,)), ((), ())),
                               preferred_element_type=jnp.float32) * s
        up = lax.dot_general(x, uw_r[0].astype(jnp.bfloat16),
                             (((1,), (0,)), ((), ())),
                             preferred_element_type=jnp.float32) * s
        act = up * gate * jax.nn.sigmoid(gate)
        o_r[...] += lax.dot_general(act.astype(jnp.bfloat16),
                                    dw_r[0].astype(jnp.bfloat16),
                                    (((1,), (0,)), ((), ())),
                                    preferred_element_type=jnp.float32)


def _ffn_call(be, used, x_sorted, s_col, gate_up_proj, down_proj):
    P, H = x_sorted.shape
    I2 = gate_up_proj.shape[2]
    J = (I2 // 2) // _IB
    G = P // _B

    def _g(g, us):
        return jnp.minimum(g, us[0] - 1)

    def _j(g, j, us):
        return jnp.where(g < us[0], j, 0)

    grid_spec = pltpu.PrefetchScalarGridSpec(
        num_scalar_prefetch=2,
        grid=(G, J),
        in_specs=[
            pl.BlockSpec((_B, H), lambda g, j, be, us: (_g(g, us), 0)),
            pl.BlockSpec((_B, 1), lambda g, j, be, us: (_g(g, us), 0)),
            pl.BlockSpec((1, H, _IB),
                         lambda g, j, be, us: (be[_g(g, us)], 0, _j(g, j, us))),
            pl.BlockSpec((1, H, _IB),
                         lambda g, j, be, us: (be[_g(g, us)], 0,
                                               _j(g, j, us) + J)),
            pl.BlockSpec((1, _IB, H),
                         lambda g, j, be, us: (be[_g(g, us)], _j(g, j, us), 0)),
        ],
        out_specs=pl.BlockSpec((_B, H), lambda g, j, be, us: (_g(g, us), 0)),
    )
    return pl.pallas_call(
        _ffn_body,
        grid_spec=grid_spec,
        out_shape=jax.ShapeDtypeStruct((P, H), jnp.float32),
        compiler_params=pltpu.CompilerParams(
            dimension_semantics=("arbitrary", "arbitrary")),
        name="moe_grouped_ffn",
    )(be, used, x_sorted, s_col, gate_up_proj, gate_up_proj, down_proj)


_SIB = 256


def _shared_body(x_r, rw_r, gw_r, uw_r, dw_r, o_r, eidx_r, score_r):
    j = pl.program_id(0)

    @pl.when(j == 0)
    def _():
        o_r[...] = jnp.zeros_like(o_r)
        logits = lax.dot_general(x_r[...], rw_r[...], (((1,), (1,)), ((), ())),
                                 preferred_element_type=jnp.float32)
        lanes = lax.broadcasted_iota(jnp.int32, logits.shape, 1)
        masked = jnp.where(lanes < _NUM_E, logits,
                           jnp.full_like(logits, -jnp.inf))
        mx = jnp.max(masked, axis=1, keepdims=True)
        ismax = jnp.logical_and(masked == mx, lanes < _NUM_E)
        eidx = jnp.min(jnp.where(ismax, lanes, 127), axis=1, keepdims=True)
        eidx_r[...] = jnp.broadcast_to(eidx, logits.shape)
        score_r[...] = jnp.broadcast_to(jax.nn.sigmoid(mx), logits.shape)

    x = x_r[...].astype(jnp.bfloat16)
    gate = lax.dot_general(x, gw_r[...].astype(jnp.bfloat16),
                           (((1,), (1,)), ((), ())),
                           preferred_element_type=jnp.float32)
    up = lax.dot_general(x, uw_r[...].astype(jnp.bfloat16),
                         (((1,), (1,)), ((), ())),
                         preferred_element_type=jnp.float32)
    act = up * gate * jax.nn.sigmoid(gate)
    o_r[...] += lax.dot_general(act.astype(jnp.bfloat16),
                                dw_r[...].astype(jnp.bfloat16),
                                (((1,), (1,)), ((), ())),
                                preferred_element_type=jnp.float32)


def _shared_call(hs, rw_pad, w_gu, w_d):
    T, H = hs.shape
    I2 = w_gu.shape[0]
    I = I2 // 2
    J = I // _SIB
    return pl.pallas_call(
        _shared_body,
        grid=(J,),
        in_specs=[
            pl.BlockSpec((T, H), lambda j: (0, 0)),
            pl.BlockSpec((_LANES, H), lambda j: (0, 0)),
            pl.BlockSpec((_SIB, H), lambda j: (j, 0)),
            pl.BlockSpec((_SIB, H), lambda j: (j + J, 0)),
            pl.BlockSpec((H, _SIB), lambda j: (0, j)),
        ],
        out_specs=[
            pl.BlockSpec((T, H), lambda j: (0, 0)),
            pl.BlockSpec((T, _LANES), lambda j: (0, 0)),
            pl.BlockSpec((T, _LANES), lambda j: (0, 0)),
        ],
        out_shape=[
            jax.ShapeDtypeStruct((T, H), jnp.float32),
            jax.ShapeDtypeStruct((T, _LANES), jnp.int32),
            jax.ShapeDtypeStruct((T, _LANES), jnp.float32),
        ],
        compiler_params=pltpu.CompilerParams(
            dimension_semantics=("arbitrary",),
            vmem_limit_bytes=112 * 1024 * 1024),
        name="moe_shared_mlp_router",
    )(hs, rw_pad, w_gu, w_gu, w_d)


def _sc_scatter_rows_call(src, idx2d, n_out):
    N, H = src.shape
    NCH, CH = idx2d.shape[0], idx2d.shape[1]
    info = plsc.get_sparse_core_info()
    NC, NS = info.num_cores, info.num_subcores
    NW = NC * NS
    ch_w = NCH // NW
    mesh = plsc.VectorSubcoreMesh(core_axis_name="c", subcore_axis_name="s")

    @functools.partial(
        pl.kernel, mesh=mesh,
        out_type=jax.ShapeDtypeStruct((n_out, H), jnp.float32),
        scratch_types=[
            pltpu.VMEM((ch_w, CH), jnp.int32),
            pltpu.VMEM((CH, H), jnp.float32),
            pltpu.VMEM((CH, H), jnp.float32),
            pltpu.SemaphoreType.DMA,
            pltpu.SemaphoreType.DMA,
            pltpu.SemaphoreType.DMA,
            pltpu.SemaphoreType.DMA,
        ],
        name="moe_sc_scatter_rows",
    )
    def k(src_hbm, idx_hbm, out_hbm, idx_v, r0, r1, sg0, sg1, ss0, ss1):
        wid = lax.axis_index("s") * NC + lax.axis_index("c")
        pltpu.sync_copy(idx_hbm.at[pl.ds(wid * ch_w, ch_w)], idx_v)
        base = wid * ch_w * CH
        bufs = (r0, r1)
        gsems = (sg0, sg1)
        ssems = (ss0, ss1)
        gcp = [None] * ch_w
        scp = [None] * ch_w
        gcp[0] = pltpu.async_copy(
            src_hbm.at[pl.ds(base, CH)], bufs[0], gsems[0])
        for c in range(ch_w):
            b = c & 1
            if c + 1 < ch_w:
                if c - 1 >= 0:
                    scp[c - 1].wait()
                gcp[c + 1] = pltpu.async_copy(
                    src_hbm.at[pl.ds(base + (c + 1) * CH, CH)],
                    bufs[b ^ 1], gsems[b ^ 1])
            gcp[c].wait()
            scp[c] = pltpu.async_copy(
                bufs[b], out_hbm.at[idx_v.at[c]], ssems[b])
        if ch_w >= 2:
            scp[ch_w - 2].wait()
        scp[ch_w - 1].wait()

    return k(src, idx2d)


def _add_body(a_r, b_r, o_r):
    o_r[...] = a_r[...] + b_r[...]


def _add_call(shared_out, ffn_tok):
    T, H = shared_out.shape
    return pl.pallas_call(
        _add_body,
        grid=(T // _B,),
        in_specs=[
            pl.BlockSpec((_B, H), lambda m: (m, 0)),
            pl.BlockSpec((_B, H), lambda m: (m, 0)),
        ],
        out_specs=pl.BlockSpec((_B, H), lambda m: (m, 0)),
        out_shape=jax.ShapeDtypeStruct((T, H), jnp.float32),
        name="moe_final_add",
    )(shared_out, ffn_tok)


def kernel(hidden_states, router_w, gate_up_proj, down_proj,
           shared_gate_up_w, shared_down_w, adapter_data, run_index):
    T, H = hidden_states.shape
    E = gate_up_proj.shape[0]
    P = T + E * _B
    G = P // _B

    rw_pad = jnp.zeros((_LANES, H), jnp.float32).at[:E, :].set(router_w)
    shared_out, eidx_w, score_w = _shared_call(hidden_states, rw_pad,
                                               shared_gate_up_w, shared_down_w)
    eidx = eidx_w[:, 0]
    score = score_w[:, 0]

    oh = (eidx[:, None] == jnp.arange(E)[None, :]).astype(jnp.int32)
    counts = jnp.sum(oh, axis=0)
    rank = jnp.sum(jnp.cumsum(oh, axis=0) * oh, axis=1) - 1
    nb = (counts + _B - 1) // _B
    cum_nb = jnp.cumsum(nb)
    base_blk = cum_nb - nb
    used = cum_nb[-1]
    pos = base_blk[eidx] * _B + rank
    tid = (T + jnp.arange(P, dtype=jnp.int32) % _B).at[pos].set(
        jnp.arange(T, dtype=jnp.int32))
    s_sorted = jnp.zeros((P,), jnp.float32).at[pos].set(score)

    g_ids = jnp.arange(G, dtype=jnp.int32)
    be = jnp.minimum(jnp.sum((g_ids[:, None] >= cum_nb[None, :]).astype(
        jnp.int32), axis=1), E - 1).astype(jnp.int32)
    used_arr = jnp.full((1,), used, jnp.int32)

    x_sorted = _sc_scatter_rows_call(hidden_states, pos.reshape(T // 16, 16), P)
    ffn_sorted = _ffn_call(be, used_arr, x_sorted,
                           s_sorted[:, None], gate_up_proj, down_proj)
    ffn_tok = _sc_scatter_rows_call(ffn_sorted, tid.reshape(P // 16, 16),
                                    T + _B)
    out = _add_call(shared_out, ffn_tok)
    return out


# speedup vs baseline: 1.1681x; 1.1681x over previous
import functools

import jax
import jax.numpy as jnp
from jax import lax
from jax.experimental import pallas as pl
from jax.experimental.pallas import tpu as pltpu
from jax.experimental.pallas import tpu_sc as plsc

_B = 256
_BF = 512
_IB = 512
_LANES = 128


_NUM_E = 8


def _ffn_body(be_r, used_r, x_r, s_r, gw_r, uw_r, dw_r, o_r):
    g = pl.program_id(0)
    j = pl.program_id(1)
    is_active = g < used_r[0]

    @pl.when(jnp.logical_and(is_active, j == 0))
    def _():
        o_r[...] = jnp.zeros_like(o_r)

    @pl.when(is_active)
    def _():
        x = x_r[...].astype(jnp.bfloat16)
        s = s_r[...]
        gate = lax.dot_general(x, gw_r[0].astype(jnp.bfloat16),
                               (((1,), (0,)), ((), ())),
                               preferred_element_type=jnp.float32) * s
        up = lax.dot_general(x, uw_r[0].astype(jnp.bfloat16),
                             (((1,), (0,)), ((), ())),
                             preferred_element_type=jnp.float32) * s
        act = up * gate * jax.nn.sigmoid(gate)
        o_r[...] += lax.dot_general(act.astype(jnp.bfloat16),
                                    dw_r[0].astype(jnp.bfloat16),
                                    (((1,), (0,)), ((), ())),
                                    preferred_element_type=jnp.float32)


def _ffn_call(be, used, x_sorted, s_col, gate_up_proj, down_proj):
    P, H = x_sorted.shape
    I2 = gate_up_proj.shape[2]
    J = (I2 // 2) // _IB
    G = P // _BF

    def _g(g, us):
        return jnp.minimum(g, us[0] - 1)

    def _j(g, j, us):
        return jnp.where(g < us[0], j, 0)

    grid_spec = pltpu.PrefetchScalarGridSpec(
        num_scalar_prefetch=2,
        grid=(G, J),
        in_specs=[
            pl.BlockSpec((_BF, H), lambda g, j, be, us: (_g(g, us), 0)),
            pl.BlockSpec((_BF, 1), lambda g, j, be, us: (_g(g, us), 0)),
            pl.BlockSpec((1, H, _IB),
                         lambda g, j, be, us: (be[_g(g, us)], 0, _j(g, j, us))),
            pl.BlockSpec((1, H, _IB),
                         lambda g, j, be, us: (be[_g(g, us)], 0,
                                               _j(g, j, us) + J)),
            pl.BlockSpec((1, _IB, H),
                         lambda g, j, be, us: (be[_g(g, us)], _j(g, j, us), 0)),
        ],
        out_specs=pl.BlockSpec((_BF, H), lambda g, j, be, us: (_g(g, us), 0)),
    )
    return pl.pallas_call(
        _ffn_body,
        grid_spec=grid_spec,
        out_shape=jax.ShapeDtypeStruct((P, H), jnp.float32),
        compiler_params=pltpu.CompilerParams(
            dimension_semantics=("arbitrary", "arbitrary")),
        name="moe_grouped_ffn",
    )(be, used, x_sorted, s_col, gate_up_proj, gate_up_proj, down_proj)


_SIB = 256


def _shared_body(x_r, rw_r, gw_r, uw_r, dw_r, o_r, eidx_r, score_r, xb_r):
    j = pl.program_id(0)

    @pl.when(j == 0)
    def _():
        o_r[...] = jnp.zeros_like(o_r)
        xb_r[...] = x_r[...].astype(jnp.bfloat16)
        logits = lax.dot_general(x_r[...], rw_r[...], (((1,), (1,)), ((), ())),
                                 preferred_element_type=jnp.float32)
        lanes = lax.broadcasted_iota(jnp.int32, logits.shape, 1)
        masked = jnp.where(lanes < _NUM_E, logits,
                           jnp.full_like(logits, -jnp.inf))
        mx = jnp.max(masked, axis=1, keepdims=True)
        ismax = jnp.logical_and(masked == mx, lanes < _NUM_E)
        eidx = jnp.min(jnp.where(ismax, lanes, 127), axis=1, keepdims=True)
        eidx_r[...] = jnp.broadcast_to(eidx, logits.shape)
        score_r[...] = jnp.broadcast_to(jax.nn.sigmoid(mx), logits.shape)

    x = xb_r[...]
    gate = lax.dot_general(x, gw_r[...].astype(jnp.bfloat16),
                           (((1,), (1,)), ((), ())),
                           preferred_element_type=jnp.float32)
    up = lax.dot_general(x, uw_r[...].astype(jnp.bfloat16),
                         (((1,), (1,)), ((), ())),
                         preferred_element_type=jnp.float32)
    act = up * gate * jax.nn.sigmoid(gate)
    o_r[...] += lax.dot_general(act.astype(jnp.bfloat16),
                                dw_r[...].astype(jnp.bfloat16),
                                (((1,), (1,)), ((), ())),
                                preferred_element_type=jnp.float32)


def _shared_call(hs, rw_pad, w_gu, w_d):
    T, H = hs.shape
    I2 = w_gu.shape[0]
    I = I2 // 2
    J = I // _SIB
    return pl.pallas_call(
        _shared_body,
        grid=(J,),
        in_specs=[
            pl.BlockSpec((T, H), lambda j: (0, 0)),
            pl.BlockSpec((_LANES, H), lambda j: (0, 0)),
            pl.BlockSpec((_SIB, H), lambda j: (j, 0)),
            pl.BlockSpec((_SIB, H), lambda j: (j + J, 0)),
            pl.BlockSpec((H, _SIB), lambda j: (0, j)),
        ],
        out_specs=[
            pl.BlockSpec((T, H), lambda j: (0, 0)),
            pl.BlockSpec((T, _LANES), lambda j: (0, 0)),
            pl.BlockSpec((T, _LANES), lambda j: (0, 0)),
        ],
        out_shape=[
            jax.ShapeDtypeStruct((T, H), jnp.float32),
            jax.ShapeDtypeStruct((T, _LANES), jnp.int32),
            jax.ShapeDtypeStruct((T, _LANES), jnp.float32),
        ],
        scratch_shapes=[pltpu.VMEM((T, H), jnp.bfloat16)],
        compiler_params=pltpu.CompilerParams(
            dimension_semantics=("arbitrary",),
            vmem_limit_bytes=112 * 1024 * 1024),
        name="moe_shared_mlp_router",
    )(hs, rw_pad, w_gu, w_gu, w_d)


def _sc_scatter_rows_call(src, idx2d, n_out):
    N, H = src.shape
    NCH, CH = idx2d.shape[0], idx2d.shape[1]
    info = plsc.get_sparse_core_info()
    NC, NS = info.num_cores, info.num_subcores
    NW = NC * NS
    ch_w = NCH // NW
    mesh = plsc.VectorSubcoreMesh(core_axis_name="c", subcore_axis_name="s")

    @functools.partial(
        pl.kernel, mesh=mesh,
        out_type=jax.ShapeDtypeStruct((n_out, H), jnp.float32),
        scratch_types=[
            pltpu.VMEM((ch_w, CH), jnp.int32),
            pltpu.VMEM((CH, H), jnp.float32),
            pltpu.VMEM((CH, H), jnp.float32),
            pltpu.SemaphoreType.DMA,
            pltpu.SemaphoreType.DMA,
            pltpu.SemaphoreType.DMA,
            pltpu.SemaphoreType.DMA,
        ],
        name="moe_sc_scatter_rows",
    )
    def k(src_hbm, idx_hbm, out_hbm, idx_v, r0, r1, sg0, sg1, ss0, ss1):
        wid = lax.axis_index("s") * NC + lax.axis_index("c")
        pltpu.sync_copy(idx_hbm.at[pl.ds(wid * ch_w, ch_w)], idx_v)
        base = wid * ch_w * CH
        bufs = (r0, r1)
        gsems = (sg0, sg1)
        ssems = (ss0, ss1)
        gcp = [None] * ch_w
        scp = [None] * ch_w
        gcp[0] = pltpu.async_copy(
            src_hbm.at[pl.ds(base, CH)], bufs[0], gsems[0])
        for c in range(ch_w):
            b = c & 1
            if c + 1 < ch_w:
                if c - 1 >= 0:
                    scp[c - 1].wait()
                gcp[c + 1] = pltpu.async_copy(
                    src_hbm.at[pl.ds(base + (c + 1) * CH, CH)],
                    bufs[b ^ 1], gsems[b ^ 1])
            gcp[c].wait()
            scp[c] = pltpu.async_copy(
                bufs[b], out_hbm.at[idx_v.at[c]], ssems[b])
        if ch_w >= 2:
            scp[ch_w - 2].wait()
        scp[ch_w - 1].wait()

    return k(src, idx2d)


def _add_body(a_r, b_r, o_r):
    o_r[...] = a_r[...] + b_r[...]


def _add_call(shared_out, ffn_tok):
    T, H = shared_out.shape
    return pl.pallas_call(
        _add_body,
        grid=(T // _B,),
        in_specs=[
            pl.BlockSpec((_B, H), lambda m: (m, 0)),
            pl.BlockSpec((_B, H), lambda m: (m, 0)),
        ],
        out_specs=pl.BlockSpec((_B, H), lambda m: (m, 0)),
        out_shape=jax.ShapeDtypeStruct((T, H), jnp.float32),
        name="moe_final_add",
    )(shared_out, ffn_tok)


def kernel(hidden_states, router_w, gate_up_proj, down_proj,
           shared_gate_up_w, shared_down_w, adapter_data, run_index):
    T, H = hidden_states.shape
    E = gate_up_proj.shape[0]
    P = T + E * _B
    G = P // _BF

    rw_pad = jnp.zeros((_LANES, H), jnp.float32).at[:E, :].set(router_w)
    shared_out, eidx_w, score_w = _shared_call(hidden_states, rw_pad,
                                               shared_gate_up_w, shared_down_w)
    eidx = eidx_w[:, 0]
    score = score_w[:, 0]

    oh = (eidx[:, None] == jnp.arange(E)[None, :]).astype(jnp.int32)
    counts = jnp.sum(oh, axis=0)
    rank = jnp.sum(jnp.cumsum(oh, axis=0) * oh, axis=1) - 1
    nb = (counts + _BF - 1) // _BF
    cum_nb = jnp.cumsum(nb)
    base_blk = cum_nb - nb
    used = cum_nb[-1]
    pos = base_blk[eidx] * _BF + rank
    tid = (T + jnp.arange(P, dtype=jnp.int32) % _B).at[pos].set(
        jnp.arange(T, dtype=jnp.int32))
    s_sorted = jnp.zeros((P,), jnp.float32).at[pos].set(score)

    g_ids = jnp.arange(G, dtype=jnp.int32)
    be = jnp.minimum(jnp.sum((g_ids[:, None] >= cum_nb[None, :]).astype(
        jnp.int32), axis=1), E - 1).astype(jnp.int32)
    used_arr = jnp.full((1,), used, jnp.int32)

    x_sorted = _sc_scatter_rows_call(hidden_states, pos.reshape(T // 16, 16), P)
    ffn_sorted = _ffn_call(be, used_arr, x_sorted,
                           s_sorted[:, None], gate_up_proj, down_proj)
    ffn_tok = _sc_scatter_rows_call(ffn_sorted, tid.reshape(P // 16, 16),
                                    T + _B)
    out = _add_call(shared_out, ffn_tok)
    return out


# speedup vs baseline: 1.1685x; 1.0003x over previous
import functools

import jax
import jax.numpy as jnp
from jax import lax
from jax.experimental import pallas as pl
from jax.experimental.pallas import tpu as pltpu
from jax.experimental.pallas import tpu_sc as plsc

_B = 256
_BF = 512
_IB = 512
_LANES = 128


_NUM_E = 8


def _ffn_body(be_r, used_r, x_r, s_r, gw_r, uw_r, dw_r, o_r):
    g = pl.program_id(0)
    j = pl.program_id(1)
    is_active = g < used_r[0]

    @pl.when(jnp.logical_and(is_active, j == 0))
    def _():
        o_r[...] = jnp.zeros_like(o_r)

    @pl.when(is_active)
    def _():
        x = x_r[...].astype(jnp.bfloat16)
        s = s_r[...]
        gate = lax.dot_general(x, gw_r[0].astype(jnp.bfloat16),
                               (((1,), (0,)), ((), ())),
                               preferred_element_type=jnp.float32) * s
        up = lax.dot_general(x, uw_r[0].astype(jnp.bfloat16),
                             (((1,), (0,)), ((), ())),
                             preferred_element_type=jnp.float32) * s
        act = up * gate * jax.nn.sigmoid(gate)
        o_r[...] += lax.dot_general(act.astype(jnp.bfloat16),
                                    dw_r[0].astype(jnp.bfloat16),
                                    (((1,), (0,)), ((), ())),
                                    preferred_element_type=jnp.float32)


def _ffn_call(be, used, x_sorted, s_col, gate_up_proj, down_proj):
    P, H = x_sorted.shape
    I2 = gate_up_proj.shape[2]
    J = (I2 // 2) // _IB
    G = P // _BF

    def _g(g, us):
        return jnp.minimum(g, us[0] - 1)

    def _j(g, j, us):
        return jnp.where(g < us[0], j, 0)

    grid_spec = pltpu.PrefetchScalarGridSpec(
        num_scalar_prefetch=2,
        grid=(G, J),
        in_specs=[
            pl.BlockSpec((_BF, H), lambda g, j, be, us: (_g(g, us), 0)),
            pl.BlockSpec((_BF, 1), lambda g, j, be, us: (_g(g, us), 0)),
            pl.BlockSpec((1, H, _IB),
                         lambda g, j, be, us: (be[_g(g, us)], 0, _j(g, j, us))),
            pl.BlockSpec((1, H, _IB),
                         lambda g, j, be, us: (be[_g(g, us)], 0,
                                               _j(g, j, us) + J)),
            pl.BlockSpec((1, _IB, H),
                         lambda g, j, be, us: (be[_g(g, us)], _j(g, j, us), 0)),
        ],
        out_specs=pl.BlockSpec((_BF, H), lambda g, j, be, us: (_g(g, us), 0)),
    )
    return pl.pallas_call(
        _ffn_body,
        grid_spec=grid_spec,
        out_shape=jax.ShapeDtypeStruct((P, H), jnp.float32),
        compiler_params=pltpu.CompilerParams(
            dimension_semantics=("arbitrary", "arbitrary"),
            vmem_limit_bytes=112 * 1024 * 1024),
        name="moe_grouped_ffn",
    )(be, used, x_sorted, s_col, gate_up_proj, gate_up_proj, down_proj)


_SIB = 256


def _shared_body(x_r, rw_r, gw_r, uw_r, dw_r, o_r, eidx_r, score_r, xb_r):
    j = pl.program_id(0)

    @pl.when(j == 0)
    def _():
        o_r[...] = jnp.zeros_like(o_r)
        xb_r[...] = x_r[...].astype(jnp.bfloat16)
        logits = lax.dot_general(x_r[...], rw_r[...], (((1,), (1,)), ((), ())),
                                 preferred_element_type=jnp.float32)
        lanes = lax.broadcasted_iota(jnp.int32, logits.shape, 1)
        masked = jnp.where(lanes < _NUM_E, logits,
                           jnp.full_like(logits, -jnp.inf))
        mx = jnp.max(masked, axis=1, keepdims=True)
        ismax = jnp.logical_and(masked == mx, lanes < _NUM_E)
        eidx = jnp.min(jnp.where(ismax, lanes, 127), axis=1, keepdims=True)
        eidx_r[...] = jnp.broadcast_to(eidx, logits.shape)
        score_r[...] = jnp.broadcast_to(jax.nn.sigmoid(mx), logits.shape)

    x = xb_r[...]
    gate = lax.dot_general(x, gw_r[...].astype(jnp.bfloat16),
                           (((1,), (1,)), ((), ())),
                           preferred_element_type=jnp.float32)
    up = lax.dot_general(x, uw_r[...].astype(jnp.bfloat16),
                         (((1,), (1,)), ((), ())),
                         preferred_element_type=jnp.float32)
    act = up * gate * jax.nn.sigmoid(gate)
    o_r[...] += lax.dot_general(act.astype(jnp.bfloat16),
                                dw_r[...].astype(jnp.bfloat16),
                                (((1,), (1,)), ((), ())),
                                preferred_element_type=jnp.float32)


def _shared_call(hs, rw_pad, w_gu, w_d):
    T, H = hs.shape
    I2 = w_gu.shape[0]
    I = I2 // 2
    J = I // _SIB
    return pl.pallas_call(
        _shared_body,
        grid=(J,),
        in_specs=[
            pl.BlockSpec((T, H), lambda j: (0, 0)),
            pl.BlockSpec((_LANES, H), lambda j: (0, 0)),
            pl.BlockSpec((_SIB, H), lambda j: (j, 0)),
            pl.BlockSpec((_SIB, H), lambda j: (j + J, 0)),
            pl.BlockSpec((H, _SIB), lambda j: (0, j)),
        ],
        out_specs=[
            pl.BlockSpec((T, H), lambda j: (0, 0)),
            pl.BlockSpec((T, _LANES), lambda j: (0, 0)),
            pl.BlockSpec((T, _LANES), lambda j: (0, 0)),
        ],
        out_shape=[
            jax.ShapeDtypeStruct((T, H), jnp.float32),
            jax.ShapeDtypeStruct((T, _LANES), jnp.int32),
            jax.ShapeDtypeStruct((T, _LANES), jnp.float32),
        ],
        scratch_shapes=[pltpu.VMEM((T, H), jnp.bfloat16)],
        compiler_params=pltpu.CompilerParams(
            dimension_semantics=("arbitrary",),
            vmem_limit_bytes=112 * 1024 * 1024),
        name="moe_shared_mlp_router",
    )(hs, rw_pad, w_gu, w_gu, w_d)


def _sc_scatter_rows_call(src, idx2d, n_out):
    N, H = src.shape
    NCH, CH = idx2d.shape[0], idx2d.shape[1]
    info = plsc.get_sparse_core_info()
    NC, NS = info.num_cores, info.num_subcores
    NW = NC * NS
    ch_w = NCH // NW
    mesh = plsc.VectorSubcoreMesh(core_axis_name="c", subcore_axis_name="s")

    @functools.partial(
        pl.kernel, mesh=mesh,
        out_type=jax.ShapeDtypeStruct((n_out, H), jnp.float32),
        scratch_types=[
            pltpu.VMEM((ch_w, CH), jnp.int32),
            pltpu.VMEM((CH, H), jnp.float32),
            pltpu.VMEM((CH, H), jnp.float32),
            pltpu.SemaphoreType.DMA,
            pltpu.SemaphoreType.DMA,
            pltpu.SemaphoreType.DMA,
            pltpu.SemaphoreType.DMA,
        ],
        name="moe_sc_scatter_rows",
    )
    def k(src_hbm, idx_hbm, out_hbm, idx_v, r0, r1, sg0, sg1, ss0, ss1):
        wid = lax.axis_index("s") * NC + lax.axis_index("c")
        pltpu.sync_copy(idx_hbm.at[pl.ds(wid * ch_w, ch_w)], idx_v)
        base = wid * ch_w * CH
        bufs = (r0, r1)
        gsems = (sg0, sg1)
        ssems = (ss0, ss1)
        gcp = [None] * ch_w
        scp = [None] * ch_w
        gcp[0] = pltpu.async_copy(
            src_hbm.at[pl.ds(base, CH)], bufs[0], gsems[0])
        for c in range(ch_w):
            b = c & 1
            if c + 1 < ch_w:
                if c - 1 >= 0:
                    scp[c - 1].wait()
                gcp[c + 1] = pltpu.async_copy(
                    src_hbm.at[pl.ds(base + (c + 1) * CH, CH)],
                    bufs[b ^ 1], gsems[b ^ 1])
            gcp[c].wait()
            scp[c] = pltpu.async_copy(
                bufs[b], out_hbm.at[idx_v.at[c]], ssems[b])
        if ch_w >= 2:
            scp[ch_w - 2].wait()
        scp[ch_w - 1].wait()

    return k(src, idx2d)


def _add_body(a_r, b_r, o_r):
    o_r[...] = a_r[...] + b_r[...]


def _add_call(shared_out, ffn_tok):
    T, H = shared_out.shape
    return pl.pallas_call(
        _add_body,
        grid=(T // _B,),
        in_specs=[
            pl.BlockSpec((_B, H), lambda m: (m, 0)),
            pl.BlockSpec((_B, H), lambda m: (m, 0)),
        ],
        out_specs=pl.BlockSpec((_B, H), lambda m: (m, 0)),
        out_shape=jax.ShapeDtypeStruct((T, H), jnp.float32),
        name="moe_final_add",
    )(shared_out, ffn_tok)


def kernel(hidden_states, router_w, gate_up_proj, down_proj,
           shared_gate_up_w, shared_down_w, adapter_data, run_index):
    T, H = hidden_states.shape
    E = gate_up_proj.shape[0]
    P = T + E * _B
    G = P // _BF

    rw_pad = jnp.zeros((_LANES, H), jnp.float32).at[:E, :].set(router_w)
    shared_out, eidx_w, score_w = _shared_call(hidden_states, rw_pad,
                                               shared_gate_up_w, shared_down_w)
    eidx = eidx_w[:, 0]
    score = score_w[:, 0]

    oh = (eidx[:, None] == jnp.arange(E)[None, :]).astype(jnp.int32)
    counts = jnp.sum(oh, axis=0)
    rank = jnp.sum(jnp.cumsum(oh, axis=0) * oh, axis=1) - 1
    nb = (counts + _BF - 1) // _BF
    cum_nb = jnp.cumsum(nb)
    base_blk = cum_nb - nb
    used = cum_nb[-1]
    pos = base_blk[eidx] * _BF + rank
    tid = (T + jnp.arange(P, dtype=jnp.int32) % _B).at[pos].set(
        jnp.arange(T, dtype=jnp.int32))
    s_sorted = jnp.zeros((P,), jnp.float32).at[pos].set(score)

    g_ids = jnp.arange(G, dtype=jnp.int32)
    be = jnp.minimum(jnp.sum((g_ids[:, None] >= cum_nb[None, :]).astype(
        jnp.int32), axis=1), E - 1).astype(jnp.int32)
    used_arr = jnp.full((1,), used, jnp.int32)

    x_sorted = _sc_scatter_rows_call(hidden_states, pos.reshape(T // 16, 16), P)
    ffn_sorted = _ffn_call(be, used_arr, x_sorted,
                           s_sorted[:, None], gate_up_proj, down_proj)
    ffn_tok = _sc_scatter_rows_call(ffn_sorted, tid.reshape(P // 16, 16),
                                    T + _B)
    out = _add_call(shared_out, ffn_tok)
    return out
